# Initial kernel scaffold; baseline (speedup 1.0000x reference)
import jax
import jax.numpy as jnp
from jax.experimental import pallas as pl


def kernel(X, pos_table):
    raise NotImplementedError("write your pallas kernel here")


# speedup vs baseline: 2.4886x; 2.4886x over previous
import jax
import jax.numpy as jnp
from jax.experimental import pallas as pl


def _add_block(x_ref, pos_ref, o_ref):
    o_ref[...] = x_ref[...] + pos_ref[...]


def kernel(X, pos_table):
    B, S, D = X.shape
    bs = 1024
    grid = (B, S // bs)
    out = pl.pallas_call(
        _add_block,
        grid=grid,
        in_specs=[
            pl.BlockSpec((1, bs, D), lambda b, s: (b, s, 0)),
            pl.BlockSpec((bs, D), lambda b, s: (s, 0)),
        ],
        out_specs=pl.BlockSpec((1, bs, D), lambda b, s: (b, s, 0)),
        out_shape=jax.ShapeDtypeStruct((B, S, D), X.dtype),
    )(X, pos_table[:S])
    return out


# speedup vs baseline: 3.1660x; 1.2722x over previous
import jax
import jax.numpy as jnp
from jax.experimental import pallas as pl


def _add_block(x_ref, pos_ref, o_ref):
    o_ref[...] = x_ref[...] + pos_ref[...]


def kernel(X, pos_table):
    B, S, D = X.shape
    bs = 1024
    grid = (S // bs, B)
    out = pl.pallas_call(
        _add_block,
        grid=grid,
        in_specs=[
            pl.BlockSpec((1, bs, D), lambda s, b: (b, s, 0)),
            pl.BlockSpec((bs, D), lambda s, b: (s, 0)),
        ],
        out_specs=pl.BlockSpec((1, bs, D), lambda s, b: (b, s, 0)),
        out_shape=jax.ShapeDtypeStruct((B, S, D), X.dtype),
    )(X, pos_table[:S])
    return out


# speedup vs baseline: 3.3033x; 1.0434x over previous
import jax
import jax.numpy as jnp
from jax.experimental import pallas as pl


def _add_block(x_ref, pos_ref, o_ref):
    o_ref[...] = x_ref[...] + pos_ref[...]


def kernel(X, pos_table):
    B, S, D = X.shape
    bs = 2048
    grid = (S // bs, B)
    out = pl.pallas_call(
        _add_block,
        grid=grid,
        in_specs=[
            pl.BlockSpec((1, bs, D), lambda s, b: (b, s, 0)),
            pl.BlockSpec((bs, D), lambda s, b: (s, 0)),
        ],
        out_specs=pl.BlockSpec((1, bs, D), lambda s, b: (b, s, 0)),
        out_shape=jax.ShapeDtypeStruct((B, S, D), X.dtype),
    )(X, pos_table[:S])
    return out
